# baseline (speedup 1.0000x reference)
import functools

import jax
import jax.numpy as jnp
from jax import lax
from jax.experimental import pallas as pl
from jax.experimental.pallas import tpu as pltpu
from jax.experimental.pallas import tpu_sc as plsc

N = 10000
E = 320000
D = 128

NC = 2
NS = 16
NW = NC * NS

E_PER_W = E // NW
CH = 80
N_CH = E_PER_W // CH
N_MAIN = (N_CH // 3) * 3 - 3
N_PAD = 10240
ROWS_PER_S = N_PAD // NS
ZR = 80
N_FLUSH = ROWS_PER_S // ZR

MM_BLK = 1000


def _mm_body(x_ref, w_ref, n_ref, o_ref):
    h = jnp.dot(x_ref[...], w_ref[...], preferred_element_type=jnp.float32)
    o_ref[...] = h * n_ref[...]


def _post_body(p_ref, n_ref, b_ref, o_ref):
    agg = p_ref[0] + p_ref[1]
    o_ref[...] = agg * n_ref[...] + b_ref[...]


def _seg_body(h_hbm, src_hbm, dst_hbm, out_hbm,
              si0, si1, si2, di0, di1, di2, rw0, rw1, rw2, acc,
              gs0, gs1, gs2, is0, is1, is2):
    c = lax.axis_index("core")
    s = lax.axis_index("subcore")
    w = c * NS + s
    base = w * E_PER_W

    sis = [si0, si1, si2]
    dis = [di0, di1, di2]
    rws = [rw0, rw1, rw2]
    gss = [gs0, gs1, gs2]
    iss = [is0, is1, is2]

    def idx_start(i, k):
        pltpu.async_copy(src_hbm.at[pl.ds(base + i * CH, CH)], sis[k], iss[k])
        pltpu.async_copy(dst_hbm.at[pl.ds(base + i * CH, CH)], dis[k], iss[k])

    def idx_wait(i, k):
        pltpu.make_async_copy(src_hbm.at[pl.ds(base + i * CH, CH)], sis[k], iss[k]).wait()
        pltpu.make_async_copy(dst_hbm.at[pl.ds(base + i * CH, CH)], dis[k], iss[k]).wait()

    def gather_start(k):
        pltpu.async_copy(h_hbm.at[sis[k]], rws[k], gss[k])

    def gather_wait(k):
        pltpu.make_async_copy(h_hbm.at[sis[k]], rws[k], gss[k]).wait()

    def scatter(k):
        pltpu.sync_copy(rws[k], acc.at[dis[k]], add=True)

    zeros16 = jnp.zeros((16,), jnp.float32)

    @pl.loop(0, ZR)
    def _(i):
        @pl.loop(0, D // 16)
        def _(j):
            rw0[i, pl.ds(j * 16, 16)] = zeros16

    idx_start(0, 0)
    idx_start(1, 1)

    zsrc = rw0.at[pl.ds(0, ZR)]

    @pl.loop(0, N_FLUSH)
    def _(t):
        pltpu.async_copy(zsrc, acc.at[pl.ds(s * ROWS_PER_S + t * ZR, ZR)], gs1)

    @pl.loop(0, N_FLUSH)
    def _(t):
        pltpu.make_async_copy(zsrc, acc.at[pl.ds(s * ROWS_PER_S + t * ZR, ZR)], gs1).wait()

    plsc.subcore_barrier()

    idx_wait(0, 0)
    gather_start(0)
    idx_wait(1, 1)
    gather_start(1)
    idx_start(2, 2)

    def step(i, k, prefetch_gather, prefetch_idx):
        k2 = (k + 2) % 3
        gather_wait(k)
        if prefetch_gather:
            idx_wait(i + 2, k2)
            gather_start(k2)
        scatter(k)
        if prefetch_idx:
            idx_start(i + 3, k)

    @pl.loop(0, N_MAIN // 3)
    def _(t):
        i0 = t * 3
        for j in range(3):
            step(i0 + j, j, True, True)

    for i in range(N_MAIN, N_CH):
        step(i, i % 3, i + 2 < N_CH, i + 3 < N_CH)

    plsc.subcore_barrier()

    for t in range(N_FLUSH):
        k = t % 2
        r = s * ROWS_PER_S + t * ZR
        rbuf = rws[k].at[pl.ds(0, ZR)]
        if t >= 2:
            rp = s * ROWS_PER_S + (t - 2) * ZR
            pltpu.make_async_copy(rbuf, out_hbm.at[pl.ds(c * N_PAD + rp, ZR)], gss[k]).wait()
        pltpu.sync_copy(acc.at[pl.ds(r, ZR)], rbuf)
        pltpu.async_copy(rbuf, out_hbm.at[pl.ds(c * N_PAD + r, ZR)], gss[k])
    for t in range(N_FLUSH - 2, N_FLUSH):
        k = t % 2
        r = s * ROWS_PER_S + t * ZR
        rbuf = rws[k].at[pl.ds(0, ZR)]
        pltpu.make_async_copy(rbuf, out_hbm.at[pl.ds(c * N_PAD + r, ZR)], gss[k]).wait()


def kernel(x, edge_index, norm, weight, bias):
    h = pl.pallas_call(
        _mm_body,
        grid=(N // MM_BLK,),
        in_specs=[
            pl.BlockSpec((MM_BLK, D), lambda i: (i, 0)),
            pl.BlockSpec((D, D), lambda i: (0, 0)),
            pl.BlockSpec((MM_BLK, 1), lambda i: (i, 0)),
        ],
        out_specs=pl.BlockSpec((MM_BLK, D), lambda i: (i, 0)),
        out_shape=jax.ShapeDtypeStruct((N, D), jnp.float32),
    )(x, weight, norm)

    src = edge_index[0]
    dst = edge_index[1]

    mesh = plsc.VectorSubcoreMesh(core_axis_name="core", subcore_axis_name="subcore")
    seg = functools.partial(
        pl.kernel,
        mesh=mesh,
        out_type=jax.ShapeDtypeStruct((NC * N_PAD, D), jnp.float32),
        scratch_types=[
            pltpu.VMEM((CH,), jnp.int32),
            pltpu.VMEM((CH,), jnp.int32),
            pltpu.VMEM((CH,), jnp.int32),
            pltpu.VMEM((CH,), jnp.int32),
            pltpu.VMEM((CH,), jnp.int32),
            pltpu.VMEM((CH,), jnp.int32),
            pltpu.VMEM((CH, D), jnp.float32),
            pltpu.VMEM((CH, D), jnp.float32),
            pltpu.VMEM((CH, D), jnp.float32),
            pltpu.VMEM_SHARED((N_PAD, D), jnp.float32),
            pltpu.SemaphoreType.DMA,
            pltpu.SemaphoreType.DMA,
            pltpu.SemaphoreType.DMA,
            pltpu.SemaphoreType.DMA,
            pltpu.SemaphoreType.DMA,
            pltpu.SemaphoreType.DMA,
        ],
    )(_seg_body)
    partial = seg(h, src, dst)
    partial = partial.reshape(NC, N_PAD, D)

    bias2d = bias.reshape(1, D)
    out = pl.pallas_call(
        _post_body,
        grid=(N // MM_BLK,),
        in_specs=[
            pl.BlockSpec((NC, MM_BLK, D), lambda i: (0, i, 0)),
            pl.BlockSpec((MM_BLK, 1), lambda i: (i, 0)),
            pl.BlockSpec((1, D), lambda i: (0, 0)),
        ],
        out_specs=pl.BlockSpec((MM_BLK, D), lambda i: (i, 0)),
        out_shape=jax.ShapeDtypeStruct((N, D), jnp.float32),
    )(partial, norm, bias2d)
    return out


# speedup vs baseline: 1.1241x; 1.1241x over previous
import functools

import jax
import jax.numpy as jnp
from jax import lax
from jax.experimental import pallas as pl
from jax.experimental.pallas import tpu as pltpu
from jax.experimental.pallas import tpu_sc as plsc

N = 10000
E = 320000
D = 128

NC = 2
NS = 16
NW = NC * NS

E_PER_W = E // NW
CH = 125
N_CH = E_PER_W // CH
IB = 16
NB = N_CH // IB
N_PAD = 10240
ROWS_PER_S = N_PAD // NS
ZR = 64
N_FLUSH = ROWS_PER_S // ZR

MM_BLK = 2000


def _mm_body(x_ref, w_ref, n_ref, o_ref):
    h = jnp.dot(x_ref[...], w_ref[...], preferred_element_type=jnp.float32)
    o_ref[...] = h * n_ref[...]


def _post_body(p_ref, n_ref, b_ref, o_ref):
    agg = p_ref[0] + p_ref[1]
    o_ref[...] = agg * n_ref[...] + b_ref[...]


def _run_block(h_hbm, acc, si, di, rows0, rows1, sem0, sem1):
    pltpu.async_copy(h_hbm.at[si.at[0]], rows0, sem0)
    pltpu.async_copy(h_hbm.at[si.at[1]], rows1, sem1)

    @pl.loop(0, IB // 2 - 1)
    def _(t):
        i = t * 2
        pltpu.make_async_copy(h_hbm.at[si.at[i]], rows0, sem0).wait()
        pltpu.sync_copy(rows0, acc.at[di.at[i]], add=True)
        pltpu.async_copy(h_hbm.at[si.at[i + 2]], rows0, sem0)
        pltpu.make_async_copy(h_hbm.at[si.at[i + 1]], rows1, sem1).wait()
        pltpu.sync_copy(rows1, acc.at[di.at[i + 1]], add=True)
        pltpu.async_copy(h_hbm.at[si.at[i + 3]], rows1, sem1)

    pltpu.make_async_copy(h_hbm.at[si.at[IB - 2]], rows0, sem0).wait()
    pltpu.sync_copy(rows0, acc.at[di.at[IB - 2]], add=True)
    pltpu.make_async_copy(h_hbm.at[si.at[IB - 1]], rows1, sem1).wait()
    pltpu.sync_copy(rows1, acc.at[di.at[IB - 1]], add=True)


def _seg_body(h_hbm, src_hbm, dst_hbm, out_hbm, src_i0, dst_i0, src_i1, dst_i1,
              rows0, rows1, acc, sem0, sem1, isem0, isem1):
    c = lax.axis_index("core")
    s = lax.axis_index("subcore")
    w = c * NS + s

    zeros16 = jnp.zeros((16,), jnp.float32)

    @pl.loop(0, ZR)
    def _(i):
        @pl.loop(0, D // 16)
        def _(j):
            rows0[i, pl.ds(j * 16, 16)] = zeros16

    pltpu.sync_copy(src_hbm.at[w, pl.ds(0, IB)], src_i0)
    pltpu.sync_copy(dst_hbm.at[w, pl.ds(0, IB)], dst_i0)

    zsrc = rows0.at[pl.ds(0, ZR)]

    @pl.loop(0, N_FLUSH)
    def _(t):
        pltpu.async_copy(zsrc, acc.at[pl.ds(s * ROWS_PER_S + t * ZR, ZR)], sem1)

    @pl.loop(0, N_FLUSH)
    def _(t):
        pltpu.make_async_copy(zsrc, acc.at[pl.ds(s * ROWS_PER_S + t * ZR, ZR)], sem1).wait()

    plsc.subcore_barrier()

    sblk = [(src_i0, dst_i0, isem0), (src_i1, dst_i1, isem1)]
    for b in range(NB):
        si, di, _ = sblk[b % 2]
        if b + 1 < NB:
            nsi, ndi, nisem = sblk[(b + 1) % 2]
            pltpu.async_copy(src_hbm.at[w, pl.ds((b + 1) * IB, IB)], nsi, nisem)
            pltpu.async_copy(dst_hbm.at[w, pl.ds((b + 1) * IB, IB)], ndi, nisem)
        _run_block(h_hbm, acc, si, di, rows0, rows1, sem0, sem1)
        if b + 1 < NB:
            pltpu.make_async_copy(src_hbm.at[w, pl.ds((b + 1) * IB, IB)], nsi, nisem).wait()
            pltpu.make_async_copy(dst_hbm.at[w, pl.ds((b + 1) * IB, IB)], ndi, nisem).wait()

    plsc.subcore_barrier()

    for t in range(N_FLUSH):
        k = t % 2
        rbuf = (rows0 if k == 0 else rows1).at[pl.ds(0, ZR)]
        sem = sem0 if k == 0 else sem1
        r = s * ROWS_PER_S + t * ZR
        if t >= 2:
            rp = s * ROWS_PER_S + (t - 2) * ZR
            pltpu.make_async_copy(rbuf, out_hbm.at[pl.ds(c * N_PAD + rp, ZR)], sem).wait()
        pltpu.sync_copy(acc.at[pl.ds(r, ZR)], rbuf)
        pltpu.async_copy(rbuf, out_hbm.at[pl.ds(c * N_PAD + r, ZR)], sem)
    for t in range(N_FLUSH - 2, N_FLUSH):
        k = t % 2
        rbuf = (rows0 if k == 0 else rows1).at[pl.ds(0, ZR)]
        sem = sem0 if k == 0 else sem1
        r = s * ROWS_PER_S + t * ZR
        pltpu.make_async_copy(rbuf, out_hbm.at[pl.ds(c * N_PAD + r, ZR)], sem).wait()


def kernel(x, edge_index, norm, weight, bias):
    h = pl.pallas_call(
        _mm_body,
        grid=(N // MM_BLK,),
        in_specs=[
            pl.BlockSpec((MM_BLK, D), lambda i: (i, 0)),
            pl.BlockSpec((D, D), lambda i: (0, 0)),
            pl.BlockSpec((MM_BLK, 1), lambda i: (i, 0)),
        ],
        out_specs=pl.BlockSpec((MM_BLK, D), lambda i: (i, 0)),
        out_shape=jax.ShapeDtypeStruct((N, D), jnp.float32),
    )(x, weight, norm)

    src = edge_index[0].reshape(NW, N_CH, CH)
    dst = edge_index[1].reshape(NW, N_CH, CH)

    mesh = plsc.VectorSubcoreMesh(core_axis_name="core", subcore_axis_name="subcore")
    seg = functools.partial(
        pl.kernel,
        mesh=mesh,
        out_type=jax.ShapeDtypeStruct((NC * N_PAD, D), jnp.float32),
        scratch_types=[
            pltpu.VMEM((IB, CH), jnp.int32),
            pltpu.VMEM((IB, CH), jnp.int32),
            pltpu.VMEM((IB, CH), jnp.int32),
            pltpu.VMEM((IB, CH), jnp.int32),
            pltpu.VMEM((CH, D), jnp.float32),
            pltpu.VMEM((CH, D), jnp.float32),
            pltpu.VMEM_SHARED((N_PAD, D), jnp.float32),
            pltpu.SemaphoreType.DMA,
            pltpu.SemaphoreType.DMA,
            pltpu.SemaphoreType.DMA,
            pltpu.SemaphoreType.DMA,
        ],
    )(_seg_body)
    partial = seg(h, src, dst)
    partial = partial.reshape(NC, N_PAD, D)

    bias2d = bias.reshape(1, D)
    out = pl.pallas_call(
        _post_body,
        grid=(N // MM_BLK,),
        in_specs=[
            pl.BlockSpec((NC, MM_BLK, D), lambda i: (0, i, 0)),
            pl.BlockSpec((MM_BLK, 1), lambda i: (i, 0)),
            pl.BlockSpec((1, D), lambda i: (0, 0)),
        ],
        out_specs=pl.BlockSpec((MM_BLK, D), lambda i: (i, 0)),
        out_shape=jax.ShapeDtypeStruct((N, D), jnp.float32),
    )(partial, norm, bias2d)
    return out
